# Initial kernel scaffold; baseline (speedup 1.0000x reference)
import jax
import jax.numpy as jnp
from jax.experimental import pallas as pl


def kernel(logits, temperatures):
    raise NotImplementedError("write your pallas kernel here")


# speedup vs baseline: 1.2309x; 1.2309x over previous
import functools

import jax
import jax.numpy as jnp
import numpy as np
from jax import lax
from jax.experimental import pallas as pl
from jax.experimental.pallas import tpu as pltpu
from jax.experimental.pallas import tpu_sc as plsc

_B = 128
_V = 100000
_NC = 2
_NS = 16
_L = 16
_NW = _NC * _NS
_RPW = _B // _NW
_CHUNK = 10000
_NCH = _V // _CHUNK
_EPS = 1e-10
_IMAX = np.int32(2147483647)

def _threefry2x32(k1, k2, x0, x1):
    x0 = x0.astype(np.uint32)
    x1 = x1.astype(np.uint32)
    ks0 = np.uint32(k1)
    ks1 = np.uint32(k2)
    ks2 = ks0 ^ ks1 ^ np.uint32(0x1BD11BDA)
    rot0 = (13, 15, 26, 6)
    rot1 = (17, 29, 16, 24)

    def rounds(x0, x1, rots):
        for r in rots:
            x0 = (x0 + x1).astype(np.uint32)
            x1 = ((x1 << np.uint32(r)) | (x1 >> np.uint32(32 - r))).astype(
                np.uint32)
            x1 = x0 ^ x1
        return x0, x1

    x0 = (x0 + ks0).astype(np.uint32)
    x1 = (x1 + ks1).astype(np.uint32)
    for i, (rots, ka, kb) in enumerate([(rot0, ks1, ks2), (rot1, ks2, ks0),
                                        (rot0, ks0, ks1), (rot1, ks1, ks2),
                                        (rot0, ks2, ks0)]):
        x0, x1 = rounds(x0, x1, rots)
        x0 = (x0 + ka).astype(np.uint32)
        x1 = (x1 + kb + np.uint32(i + 1)).astype(np.uint32)
    return x0, x1


def _noise_log() -> np.ndarray:
    kk = _threefry2x32(0, 0, np.zeros(1, np.uint32),
                       np.array([12345], np.uint32))
    k1, k2 = np.uint32(kk[0][0]), np.uint32(kk[1][0])
    n = _B * _V
    i64 = np.arange(n, dtype=np.uint64)
    c1 = (i64 >> np.uint64(32)).astype(np.uint32)
    c2 = (i64 & np.uint64(0xFFFFFFFF)).astype(np.uint32)
    b1, b2 = _threefry2x32(k1, k2, c1, c2)
    bits = b1 ^ b2
    flo = ((bits >> np.uint32(9)) | np.uint32(0x3F800000)).view(np.float32)
    u = (flo - np.float32(1.0)).astype(np.float32)
    noise = (-np.log1p(-u)).astype(np.float32)
    return np.log(noise + np.float32(_EPS)).astype(np.float32).reshape(_B, _V)


_NOISE_LOG = _noise_log()

_mesh = plsc.VectorSubcoreMesh(
    core_axis_name="c", subcore_axis_name="s", num_cores=_NC, num_subcores=_NS
)


@functools.partial(
    pl.kernel,
    out_type=jax.ShapeDtypeStruct((_NW * _L,), jnp.int32),
    mesh=_mesh,
    compiler_params=pltpu.CompilerParams(needs_layout_passes=False),
    scratch_types=[
        pltpu.VMEM((_CHUNK,), jnp.float32),
        pltpu.VMEM((_CHUNK,), jnp.float32),
        pltpu.VMEM((_CHUNK,), jnp.float32),
        pltpu.VMEM((_CHUNK,), jnp.float32),
        pltpu.VMEM((_L,), jnp.float32),
        pltpu.VMEM((_L,), jnp.int32),
        pltpu.SemaphoreType.DMA,
        pltpu.SemaphoreType.DMA,
    ],
)
def _sampler(logits_hbm, nlog_hbm, tb_hbm, out_hbm,
             lb0, nb0, lb1, nb1, tb, rb, sem0, sem1):
    w = lax.axis_index("s") * _NC + lax.axis_index("c")
    lane = lax.iota(jnp.int32, _L)

    bufs = ((lb0, nb0, sem0), (lb1, nb1, sem1))
    ntask = _RPW * _NCH

    def fire(t):
        row = w * _RPW + (t // _NCH)
        off = row * _V + (t % _NCH) * _CHUNK
        lb, nb, sem = bufs[t % 2]
        h1 = pltpu.async_copy(logits_hbm.at[pl.ds(off, _CHUNK)], lb, sem)
        h2 = pltpu.async_copy(nlog_hbm.at[pl.ds(off, _CHUNK)], nb, sem)
        return (h1, h2)

    handles = [None] * ntask
    handles[0] = fire(0)
    rvec = jnp.zeros((_L,), jnp.int32)
    bs = jnp.full((_L,), -jnp.inf, jnp.float32)
    bsi = jnp.zeros((_L,), jnp.int32)
    tvec = jnp.zeros((_L,), jnp.float32)

    for t in range(ntask):
        j, c = t // _NCH, t % _NCH
        if c == 0:
            bs = jnp.full((_L,), -jnp.inf, jnp.float32)
            bsi = jnp.zeros((_L,), jnp.int32)
            pltpu.sync_copy(tb_hbm.at[pl.ds((w * _RPW + j) * _L, _L)], tb)
            tvec = tb[...]
        if t + 1 < ntask:
            handles[t + 1] = fire(t + 1)
        h1, h2 = handles[t]
        h1.wait()
        h2.wait()
        lb, nb, _ = bufs[t % 2]
        base = c * _CHUNK

        def body(i, carry, lb=lb, nb=nb, base=base, tv=tvec):
            vbs, vbsi = carry
            off = i * _L
            lv = lb[pl.ds(off, _L)]
            nv = nb[pl.ds(off, _L)]
            s = lv - tv * nv
            idx = lane + (base + off)
            upd = s > vbs
            return jnp.where(upd, s, vbs), jnp.where(upd, idx, vbsi)

        bs, bsi = lax.fori_loop(0, _CHUNK // _L, body, (bs, bsi))

        if c == _NCH - 1:
            m = jnp.max(bs)
            tok = jnp.min(jnp.where(bs == m, bsi, _IMAX))
            rvec = jnp.where(lane == j, tok, rvec)

    rb[...] = rvec
    pltpu.sync_copy(rb, out_hbm.at[pl.ds(w * _L, _L)])


def kernel(logits, temperatures):
    tb = jnp.broadcast_to(temperatures[:, None], (_B, _L)).reshape(_B * _L)
    flat = _sampler(logits.astype(jnp.float32).reshape(_B * _V),
                    jnp.asarray(_NOISE_LOG).reshape(_B * _V), tb)
    return flat.reshape(_NW, _L)[:, :_RPW].reshape(_B)


# speedup vs baseline: 1.5066x; 1.2239x over previous
import functools

import jax
import jax.numpy as jnp
import numpy as np
from jax import lax
from jax.experimental import pallas as pl
from jax.experimental.pallas import tpu as pltpu
from jax.experimental.pallas import tpu_sc as plsc

_B = 128
_V = 100000
_NC = 2
_NS = 16
_L = 16
_NW = _NC * _NS
_RPW = _B // _NW
_CHUNK = 20000
_NCH = _V // _CHUNK
_UNR = 5
_EPS = 1e-10
_IMAX = np.int32(2147483647)

def _threefry2x32(k1, k2, x0, x1):
    x0 = x0.astype(np.uint32)
    x1 = x1.astype(np.uint32)
    ks0 = np.uint32(k1)
    ks1 = np.uint32(k2)
    ks2 = ks0 ^ ks1 ^ np.uint32(0x1BD11BDA)
    rot0 = (13, 15, 26, 6)
    rot1 = (17, 29, 16, 24)

    def rounds(x0, x1, rots):
        for r in rots:
            x0 = (x0 + x1).astype(np.uint32)
            x1 = ((x1 << np.uint32(r)) | (x1 >> np.uint32(32 - r))).astype(
                np.uint32)
            x1 = x0 ^ x1
        return x0, x1

    x0 = (x0 + ks0).astype(np.uint32)
    x1 = (x1 + ks1).astype(np.uint32)
    for i, (rots, ka, kb) in enumerate([(rot0, ks1, ks2), (rot1, ks2, ks0),
                                        (rot0, ks0, ks1), (rot1, ks1, ks2),
                                        (rot0, ks2, ks0)]):
        x0, x1 = rounds(x0, x1, rots)
        x0 = (x0 + ka).astype(np.uint32)
        x1 = (x1 + kb + np.uint32(i + 1)).astype(np.uint32)
    return x0, x1


def _noise_log() -> np.ndarray:
    kk = _threefry2x32(0, 0, np.zeros(1, np.uint32),
                       np.array([12345], np.uint32))
    k1, k2 = np.uint32(kk[0][0]), np.uint32(kk[1][0])
    n = _B * _V
    i64 = np.arange(n, dtype=np.uint64)
    c1 = (i64 >> np.uint64(32)).astype(np.uint32)
    c2 = (i64 & np.uint64(0xFFFFFFFF)).astype(np.uint32)
    b1, b2 = _threefry2x32(k1, k2, c1, c2)
    bits = b1 ^ b2
    flo = ((bits >> np.uint32(9)) | np.uint32(0x3F800000)).view(np.float32)
    u = (flo - np.float32(1.0)).astype(np.float32)
    noise = (-np.log1p(-u)).astype(np.float32)
    return np.log(noise + np.float32(_EPS)).astype(np.float32).reshape(_B, _V)


_NOISE_LOG = _noise_log()

_mesh = plsc.VectorSubcoreMesh(
    core_axis_name="c", subcore_axis_name="s", num_cores=_NC, num_subcores=_NS
)


@functools.partial(
    pl.kernel,
    out_type=jax.ShapeDtypeStruct((_NW * _L,), jnp.int32),
    mesh=_mesh,
    compiler_params=pltpu.CompilerParams(needs_layout_passes=False),
    scratch_types=[
        pltpu.VMEM((_CHUNK,), jnp.float32),
        pltpu.VMEM((_CHUNK,), jnp.float32),
        pltpu.VMEM((_CHUNK,), jnp.float32),
        pltpu.VMEM((_CHUNK,), jnp.float32),
        pltpu.VMEM((_L,), jnp.float32),
        pltpu.VMEM((_L,), jnp.int32),
        pltpu.SemaphoreType.DMA,
        pltpu.SemaphoreType.DMA,
    ],
)
def _sampler(logits_hbm, nlog_hbm, tb_hbm, out_hbm,
             lb0, nb0, lb1, nb1, tb, rb, sem0, sem1):
    w = lax.axis_index("s") * _NC + lax.axis_index("c")
    lane = lax.iota(jnp.int32, _L)

    bufs = ((lb0, nb0, sem0), (lb1, nb1, sem1))
    ntask = _RPW * _NCH

    def fire(t):
        row = w * _RPW + (t // _NCH)
        off = row * _V + (t % _NCH) * _CHUNK
        lb, nb, sem = bufs[t % 2]
        h1 = pltpu.async_copy(logits_hbm.at[pl.ds(off, _CHUNK)], lb, sem)
        h2 = pltpu.async_copy(nlog_hbm.at[pl.ds(off, _CHUNK)], nb, sem)
        return (h1, h2)

    handles = [None] * ntask
    handles[0] = fire(0)
    rvec = jnp.zeros((_L,), jnp.int32)
    neg_inf = jnp.full((_L,), -jnp.inf, jnp.float32)
    zeros_i = jnp.zeros((_L,), jnp.int32)
    bs = (neg_inf,) * _UNR
    bsi = (zeros_i,) * _UNR
    tvec = jnp.zeros((_L,), jnp.float32)

    for t in range(ntask):
        j, c = t // _NCH, t % _NCH
        if c == 0:
            bs = (neg_inf,) * _UNR
            bsi = (zeros_i,) * _UNR
            pltpu.sync_copy(tb_hbm.at[pl.ds((w * _RPW + j) * _L, _L)], tb)
            tvec = tb[...]
        if t + 1 < ntask:
            handles[t + 1] = fire(t + 1)
        h1, h2 = handles[t]
        h1.wait()
        h2.wait()
        lb, nb, _ = bufs[t % 2]
        base = c * _CHUNK

        def body(i, carry, lb=lb, nb=nb, base=base, tv=tvec):
            vbs = list(carry[:_UNR])
            vbsi = list(carry[_UNR:])
            off = i * (_L * _UNR)
            for k in range(_UNR):
                koff = off + k * _L
                lv = lb[pl.ds(koff, _L)]
                nv = nb[pl.ds(koff, _L)]
                s = lv - tv * nv
                idx = lane + (base + koff)
                upd = s > vbs[k]
                vbs[k] = jnp.where(upd, s, vbs[k])
                vbsi[k] = jnp.where(upd, idx, vbsi[k])
            return tuple(vbs) + tuple(vbsi)

        out = lax.fori_loop(0, _CHUNK // (_L * _UNR), body, bs + bsi)
        bs, bsi = out[:_UNR], out[_UNR:]

        if c == _NCH - 1:
            mbs, mbsi = bs[0], bsi[0]
            for k in range(1, _UNR):
                take = (bs[k] > mbs) | ((bs[k] == mbs) & (bsi[k] < mbsi))
                mbs = jnp.where(take, bs[k], mbs)
                mbsi = jnp.where(take, bsi[k], mbsi)
            m = jnp.max(mbs)
            tok = jnp.min(jnp.where(mbs == m, mbsi, _IMAX))
            rvec = jnp.where(lane == j, tok, rvec)

    rb[...] = rvec
    pltpu.sync_copy(rb, out_hbm.at[pl.ds(w * _L, _L)])


def kernel(logits, temperatures):
    tb = jnp.broadcast_to(temperatures[:, None], (_B, _L)).reshape(_B * _L)
    flat = _sampler(logits.astype(jnp.float32).reshape(_B * _V),
                    jnp.asarray(_NOISE_LOG).reshape(_B * _V), tb)
    return flat.reshape(_NW, _L)[:, :_RPW].reshape(_B)


# speedup vs baseline: 2.0771x; 1.3787x over previous
import functools

import jax
import jax.numpy as jnp
import numpy as np
from jax import lax
from jax.experimental import pallas as pl
from jax.experimental.pallas import tpu as pltpu
from jax.experimental.pallas import tpu_sc as plsc

_B = 128
_V = 100000
_NC = 2
_NS = 16
_L = 16
_NW = _NC * _NS
_NBLK = 16
_RB = 8
_TILE = 128
_NT_FULL = _V // _TILE
_CT = 11
_NCH = 36
_HOFF = 385
_CW = _CT * _TILE
_TRK = 4
_EPS = 1e-10
_IMAX = np.int32(2147483647)


def _threefry2x32(k1, k2, x0, x1):
    x0 = x0.astype(np.uint32)
    x1 = x1.astype(np.uint32)
    ks0 = np.uint32(k1)
    ks1 = np.uint32(k2)
    ks2 = ks0 ^ ks1 ^ np.uint32(0x1BD11BDA)
    rot0 = (13, 15, 26, 6)
    rot1 = (17, 29, 16, 24)

    def rounds(x0, x1, rots):
        for r in rots:
            x0 = (x0 + x1).astype(np.uint32)
            x1 = ((x1 << np.uint32(r)) | (x1 >> np.uint32(32 - r))).astype(
                np.uint32)
            x1 = x0 ^ x1
        return x0, x1

    x0 = (x0 + ks0).astype(np.uint32)
    x1 = (x1 + ks1).astype(np.uint32)
    for i, (rots, ka, kb) in enumerate([(rot0, ks1, ks2), (rot1, ks2, ks0),
                                        (rot0, ks0, ks1), (rot1, ks1, ks2),
                                        (rot0, ks2, ks0)]):
        x0, x1 = rounds(x0, x1, rots)
        x0 = (x0 + ka).astype(np.uint32)
        x1 = (x1 + kb + np.uint32(i + 1)).astype(np.uint32)
    return x0, x1


def _noise_log() -> np.ndarray:
    kk = _threefry2x32(0, 0, np.zeros(1, np.uint32),
                       np.array([12345], np.uint32))
    k1, k2 = np.uint32(kk[0][0]), np.uint32(kk[1][0])
    n = _B * _V
    i64 = np.arange(n, dtype=np.uint64)
    c1 = (i64 >> np.uint64(32)).astype(np.uint32)
    c2 = (i64 & np.uint64(0xFFFFFFFF)).astype(np.uint32)
    b1, b2 = _threefry2x32(k1, k2, c1, c2)
    bits = b1 ^ b2
    flo = ((bits >> np.uint32(9)) | np.uint32(0x3F800000)).view(np.float32)
    u = (flo - np.float32(1.0)).astype(np.float32)
    noise = (-np.log1p(-u)).astype(np.float32)
    return np.log(noise + np.float32(_EPS)).astype(np.float32).reshape(_B, _V)


_NOISE_LOG = _noise_log()

_mesh = plsc.VectorSubcoreMesh(
    core_axis_name="c", subcore_axis_name="s", num_cores=_NC, num_subcores=_NS
)


@functools.partial(
    pl.kernel,
    out_type=(
        jax.ShapeDtypeStruct((_NW * _L,), jnp.float32),
        jax.ShapeDtypeStruct((_NW * _L,), jnp.int32),
    ),
    mesh=_mesh,
    compiler_params=pltpu.CompilerParams(needs_layout_passes=False),
    scratch_types=[
        pltpu.VMEM((_RB, _CW), jnp.float32),
        pltpu.VMEM((_RB, _CW), jnp.float32),
        pltpu.VMEM((_RB, _CW), jnp.float32),
        pltpu.VMEM((_RB, _CW), jnp.float32),
        pltpu.VMEM((_RB * _L,), jnp.float32),
        pltpu.VMEM((_RB * _L,), jnp.float32),
        pltpu.VMEM((_RB * _L,), jnp.int32),
        pltpu.VMEM((_L,), jnp.float32),
        pltpu.VMEM((_L,), jnp.int32),
        pltpu.SemaphoreType.DMA,
        pltpu.SemaphoreType.DMA,
        pltpu.SemaphoreType.DMA,
    ],
)
def _sampler(logits_hbm, nlog_hbm, tb_hbm, oval_hbm, oidx_hbm,
             lb0, nb0, lb1, nb1, tv_v, sbv_v, sbi_v, rv_v, ri_v,
             sem0, sem1, sem2):
    w = lax.axis_index("s") * _NC + lax.axis_index("c")
    blk = w // 2
    half = w % 2
    rowbase = pl.multiple_of(blk * _RB, _RB)
    colbase = half * (_HOFF * _TILE)
    lane = lax.iota(jnp.int32, _L)
    neg_inf = jnp.full((_L,), -jnp.inf, jnp.float32)
    zeros_i = jnp.zeros((_L,), jnp.int32)

    pltpu.async_copy(
        tb_hbm.at[pl.ds(blk * _RB * _L, _RB * _L)], tv_v, sem2).wait()
    for r in range(_RB):
        sbv_v[pl.ds(r * _L, _L)] = neg_inf
        sbi_v[pl.ds(r * _L, _L)] = zeros_i

    bufs = ((lb0, nb0, sem0), (lb1, nb1, sem1))

    def fire(c):
        col = pl.multiple_of(colbase + c * _CW, _TILE)
        lb, nb, sem = bufs[c % 2]
        h1 = pltpu.async_copy(
            logits_hbm.at[pl.ds(rowbase, _RB), pl.ds(col, _CW)], lb, sem)
        h2 = pltpu.async_copy(
            nlog_hbm.at[pl.ds(rowbase, _RB), pl.ds(col, _CW)], nb, sem)
        return (h1, h2)

    handles = [None] * _NCH
    handles[0] = fire(0)
    for c in range(_NCH):
        if c + 1 < _NCH:
            handles[c + 1] = fire(c + 1)
        h1, h2 = handles[c]
        h1.wait()
        h2.wait()
        lb, nb, _ = bufs[c % 2]
        ccol = colbase + c * _CW

        def rbody(r, _, lb=lb, nb=nb, ccol=ccol):
            tvec = tv_v[pl.ds(r * _L, _L)]
            bs = [neg_inf] * _TRK
            bsi = [zeros_i] * _TRK

            def tbody(t, carry, lb=lb, nb=nb, r=r, ccol=ccol, tvec=tvec):
                vbs = list(carry[:_TRK])
                vbsi = list(carry[_TRK:])
                toff = t * _TILE
                for k in range(_TILE // _L):
                    koff = toff + k * _L
                    lv = lb[r, pl.ds(koff, _L)]
                    nv = nb[r, pl.ds(koff, _L)]
                    s = lv - tvec * nv
                    idx = lane + (ccol + koff)
                    kk = k % _TRK
                    upd = s > vbs[kk]
                    vbs[kk] = jnp.where(upd, s, vbs[kk])
                    vbsi[kk] = jnp.where(upd, idx, vbsi[kk])
                return tuple(vbs) + tuple(vbsi)

            out = lax.fori_loop(0, _CT, tbody, tuple(bs) + tuple(bsi))
            mb, mi = out[0], out[_TRK]
            for k in range(1, _TRK):
                ob, oi = out[k], out[_TRK + k]
                take = (ob > mb) | ((ob == mb) & (oi < mi))
                mb = jnp.where(take, ob, mb)
                mi = jnp.where(take, oi, mi)
            pb = sbv_v[pl.ds(r * _L, _L)]
            pi = sbi_v[pl.ds(r * _L, _L)]
            take = (mb > pb) | ((mb == pb) & (mi < pi))
            sbv_v[pl.ds(r * _L, _L)] = jnp.where(take, mb, pb)
            sbi_v[pl.ds(r * _L, _L)] = jnp.where(take, mi, pi)
            return 0

        lax.fori_loop(0, _RB, rbody, 0)

    def mbody(r, carry):
        rv, ri = carry
        pb = sbv_v[pl.ds(r * _L, _L)]
        pi = sbi_v[pl.ds(r * _L, _L)]
        m = jnp.max(pb)
        tok = jnp.min(jnp.where(pb == m, pi, _IMAX))
        sel = lane == r
        return jnp.where(sel, m, rv), jnp.where(sel, tok, ri)

    rv, ri = lax.fori_loop(0, _RB, mbody, (neg_inf, zeros_i))
    rv_v[...] = rv
    ri_v[...] = ri
    pltpu.async_copy(rv_v, oval_hbm.at[pl.ds(w * _L, _L)], sem2).wait()
    pltpu.async_copy(ri_v, oidx_hbm.at[pl.ds(w * _L, _L)], sem2).wait()


def kernel(logits, temperatures):
    logits = logits.astype(jnp.float32)
    nlog = jnp.asarray(_NOISE_LOG)
    tb = jnp.broadcast_to(temperatures[:, None], (_B, _L)).reshape(_B * _L)
    vals, idxs = _sampler(logits, nlog, tb)
    kv2 = vals.reshape(_NBLK, 2, _L)[:, :, :_RB]
    ki2 = idxs.reshape(_NBLK, 2, _L)[:, :, :_RB]
    cand_v = jnp.stack(
        [kv2[:, 0].reshape(_B), kv2[:, 1].reshape(_B)], axis=1)
    cand_i = jnp.stack(
        [ki2[:, 0].reshape(_B), ki2[:, 1].reshape(_B)], axis=1)
    tail0 = _NT_FULL * _TILE
    st = logits[:, tail0:] - temperatures[:, None] * nlog[:, tail0:]
    tv = jnp.max(st, axis=1)
    ti = (tail0 + jnp.argmax(st, axis=1)).astype(jnp.int32)
    cand_v = jnp.concatenate([cand_v, tv[:, None]], axis=1)
    cand_i = jnp.concatenate([cand_i, ti[:, None]], axis=1)
    mv, mi = cand_v[:, 0], cand_i[:, 0]
    for k in range(1, 3):
        ov, oi = cand_v[:, k], cand_i[:, k]
        take = (ov > mv) | ((ov == mv) & (oi < mi))
        mv = jnp.where(take, ov, mv)
        mi = jnp.where(take, oi, mi)
    return mi
